# Initial kernel scaffold; baseline (speedup 1.0000x reference)
import jax
import jax.numpy as jnp
from jax.experimental import pallas as pl


def kernel(tokens, embedding, position_embedding):
    raise NotImplementedError("write your pallas kernel here")


# speedup vs baseline: 3.2550x; 3.2550x over previous
import functools

import jax
import jax.numpy as jnp
from jax import lax
from jax.experimental import pallas as pl
from jax.experimental.pallas import tpu as pltpu
from jax.experimental.pallas import tpu_sc as plsc

_NUM_TOKENS = 1024
_P = 56
_D = 64
_B = 16384
_N_ROWS = _B * _P
_NW = 32
_ROWS_PER_W = _N_ROWS // _NW
_CHUNK = 512
_N_CHUNKS = _ROWS_PER_W // _CHUNK
_VPC = _CHUNK // 16


def _fuse_body(emb_ref, pos_ref, out_ref):
    out_ref[...] = emb_ref[...][:, None, :] + pos_ref[...][None, :, :]


def _build_fused(embedding, position_embedding):
    fused = pl.pallas_call(
        _fuse_body,
        out_shape=jax.ShapeDtypeStruct((_NUM_TOKENS, _P, _D), jnp.float32),
        grid=(8,),
        in_specs=[
            pl.BlockSpec((_NUM_TOKENS // 8, _D), lambda i: (i, 0)),
            pl.BlockSpec((_P, _D), lambda i: (0, 0)),
        ],
        out_specs=pl.BlockSpec((_NUM_TOKENS // 8, _P, _D), lambda i: (i, 0, 0)),
    )(embedding, position_embedding)
    return fused.reshape(_NUM_TOKENS * _P, _D)


_sc_mesh = plsc.VectorSubcoreMesh(core_axis_name="c", subcore_axis_name="s")


@functools.partial(
    pl.kernel,
    mesh=_sc_mesh,
    out_type=jax.ShapeDtypeStruct((_N_ROWS, _D), jnp.float32),
    scratch_types=[
        pltpu.VMEM((_CHUNK,), jnp.int32),
        pltpu.VMEM((_CHUNK,), jnp.int32),
        pltpu.VMEM((4, 128), jnp.int32),
        pltpu.VMEM((_CHUNK, _D), jnp.float32),
        pltpu.SemaphoreType.DMA,
    ],
    compiler_params=pltpu.CompilerParams(use_tc_tiling_on_sc=False),
)
def _sc_embed(tok_hbm, fused_hbm, out_hbm, tok_v, pos0_v, idx_v, rows_v, sem):
    wid = lax.axis_index("s") * 2 + lax.axis_index("c")
    for i in range(_VPC):
        lanes = lax.iota(jnp.int32, 16) + (i * 16)
        pos0_v[pl.ds(i * 16, 16)] = lax.rem(lanes, _P)

    def body(j, carry):
        base = wid * _ROWS_PER_W + j * _CHUNK
        pltpu.sync_copy(tok_hbm.at[pl.ds(base, _CHUNK)], tok_v)
        sj = lax.rem(j * (_CHUNK % _P), _P)
        sjv = jnp.full((16,), sj, dtype=jnp.int32)
        for i in range(_VPC):
            t16 = tok_v[pl.ds(i * 16, 16)]
            po = lax.rem(pos0_v[pl.ds(i * 16, 16)] + sjv, _P)
            comb = t16 * _P + po
            idx_v[i // 8, pl.ds((i % 8) * 16, 16)] = comb
        cps = [
            pltpu.async_copy(
                fused_hbm.at[idx_v.at[q]],
                rows_v.at[pl.ds(q * 128, 128)],
                sem,
            )
            for q in range(4)
        ]
        for cp in cps:
            cp.wait()
        pltpu.sync_copy(rows_v, out_hbm.at[pl.ds(base, _CHUNK)])
        return carry

    lax.fori_loop(0, _N_CHUNKS, body, 0)


def kernel(tokens, embedding, position_embedding):
    tok_flat = tokens.reshape(-1).astype(jnp.int32)
    fused = _build_fused(embedding, position_embedding)
    out_flat = _sc_embed(tok_flat, fused)
    return out_flat.reshape(_B, _P, _D)


# speedup vs baseline: 4.3150x; 1.3256x over previous
import functools

import jax
import jax.numpy as jnp
from jax import lax
from jax.experimental import pallas as pl
from jax.experimental.pallas import tpu as pltpu
from jax.experimental.pallas import tpu_sc as plsc

_NUM_TOKENS = 1024
_P = 56
_D = 64
_B = 16384
_N_ROWS = _B * _P
_NW = 32
_ROWS_PER_W = _N_ROWS // _NW
_IDX_W = 128
_IDX_ROWS = _ROWS_PER_W // _IDX_W
_CHUNK = 512
_Q = _CHUNK // _IDX_W
_N_CHUNKS = _ROWS_PER_W // _CHUNK


def _fuse_body(emb_ref, pos_ref, out_ref):
    out_ref[...] = emb_ref[...][:, None, :] + pos_ref[...][None, :, :]


def _build_fused(embedding, position_embedding):
    fused = pl.pallas_call(
        _fuse_body,
        out_shape=jax.ShapeDtypeStruct((_NUM_TOKENS, _P, _D), jnp.float32),
        grid=(8,),
        in_specs=[
            pl.BlockSpec((_NUM_TOKENS // 8, _D), lambda i: (i, 0)),
            pl.BlockSpec((_P, _D), lambda i: (0, 0)),
        ],
        out_specs=pl.BlockSpec((_NUM_TOKENS // 8, _P, _D), lambda i: (i, 0, 0)),
    )(embedding, position_embedding)
    return fused.reshape(_NUM_TOKENS * _P, _D)


_sc_mesh = plsc.VectorSubcoreMesh(core_axis_name="c", subcore_axis_name="s")


@functools.partial(
    pl.kernel,
    mesh=_sc_mesh,
    out_type=jax.ShapeDtypeStruct((_N_ROWS, _D), jnp.float32),
    scratch_types=[
        pltpu.VMEM((_IDX_ROWS, _IDX_W), jnp.int32),
        pltpu.VMEM((2, _CHUNK, _D), jnp.float32),
        pltpu.SemaphoreType.DMA,
        pltpu.SemaphoreType.DMA,
        pltpu.SemaphoreType.DMA,
        pltpu.SemaphoreType.DMA,
    ],
    compiler_params=pltpu.CompilerParams(use_tc_tiling_on_sc=False),
)
def _sc_embed(tok_hbm, fused_hbm, out_hbm, idx_v, rows_v,
              gsem0, gsem1, osem0, osem1):
    wid = lax.axis_index("s") * 2 + lax.axis_index("c")
    gsem = (gsem0, gsem1)
    osem = (osem0, osem1)

    pltpu.sync_copy(tok_hbm.at[pl.ds(wid * _IDX_ROWS, _IDX_ROWS)], idx_v)

    def idx_body(r, carry):
        row = idx_v.at[r]
        rbase = r * _IDX_W
        for i in range(_IDX_W // 16):
            off = i * 16
            k16 = jnp.full((16,), rbase + off, dtype=jnp.int32) + lax.iota(
                jnp.int32, 16)
            po = lax.rem(k16, _P)
            row[pl.ds(off, 16)] = row[pl.ds(off, 16)] * _P + po
        return carry

    lax.fori_loop(0, _IDX_ROWS, idx_body, 0)

    def fire_gathers(j, b):
        for q in range(_Q):
            pltpu.async_copy(
                fused_hbm.at[idx_v.at[j * _Q + q]],
                rows_v.at[b].at[pl.ds(q * _IDX_W, _IDX_W)],
                gsem[b],
            )

    def wait_g(b):
        pltpu.make_async_copy(
            fused_hbm.at[pl.ds(0, _CHUNK)], rows_v.at[b], gsem[b]).wait()

    def fire_out(j, b):
        base = wid * _ROWS_PER_W + j * _CHUNK
        pltpu.async_copy(rows_v.at[b], out_hbm.at[pl.ds(base, _CHUNK)],
                         osem[b])

    def wait_o(b):
        pltpu.make_async_copy(
            rows_v.at[b], out_hbm.at[pl.ds(0, _CHUNK)], osem[b]).wait()

    fire_gathers(0, 0)

    def body(jj, carry):
        j0 = 2 * jj

        wait_g(0)
        fire_out(j0, 0)

        @pl.when(jj >= 1)
        def _():
            wait_o(1)
        fire_gathers(j0 + 1, 1)

        wait_g(1)
        fire_out(j0 + 1, 1)

        wait_o(0)

        @pl.when(jj <= _N_CHUNKS // 2 - 2)
        def _():
            fire_gathers(j0 + 2, 0)

        return carry

    lax.fori_loop(0, _N_CHUNKS // 2, body, 0)
    wait_o(1)


def kernel(tokens, embedding, position_embedding):
    tok2d = tokens.reshape(_N_ROWS // _IDX_W, _IDX_W).astype(jnp.int32)
    fused = _build_fused(embedding, position_embedding)
    out_flat = _sc_embed(tok2d, fused)
    return out_flat.reshape(_B, _P, _D)
